# baseline (speedup 1.0000x reference)
import functools

import jax
import jax.numpy as jnp
from jax import lax
from jax.experimental import pallas as pl
from jax.experimental.pallas import tpu as pltpu
from jax.experimental.pallas import tpu_sc as plsc

NC = 2
NS = 16
NW = NC * NS
LANES = 16
CB = 16


def _sc_body(B, N, D, cw_hbm, xw_hbm, nw_hbm, ctab_hbm, xtab_hbm, out_hbm,
             *scr):
  T = N + 1
  TPW = 64
  orows = CB * TPW // 128
  bpw = B // NW
  nchunk = bpw // CB
  nreg = D // LANES
  cwa, xwa = scr[0:2]
  slots = [scr[2:8], scr[8:14]]
  sem_i = scr[14:16]
  sem_g = scr[16:18]
  sem_o = scr[18:20]
  lanes = lax.iota(jnp.int32, LANES)

  wid = lax.axis_index("s") * NC + lax.axis_index("c")
  base = wid * bpw

  def nw_stage_pair(slot, c):
    gbase = base + c * CB
    return (nw_hbm.at[pl.ds(gbase, CB)], slots[slot][0])

  def repack(slot):
    nw2c, nwf = slots[slot][0:2]
    for g in range(CB * N // LANES):
      flat = g * LANES + lanes
      nwf[pl.ds(g * LANES, LANES)] = plsc.load_gather(
          nw2c, [flat // N, flat % N])

  def gather_pairs(slot, c):
    nwf, cen, ctx, neg = slots[slot][1:5]
    cb = c * CB
    return [
        (ctab_hbm.at[cwa.at[pl.ds(cb, CB)]], cen),
        (xtab_hbm.at[xwa.at[pl.ds(cb, CB)]], ctx),
        (xtab_hbm.at[nwf.at[pl.ds(0, 128)]], neg.at[pl.ds(0, 128)]),
        (xtab_hbm.at[nwf.at[pl.ds(128, 128)]], neg.at[pl.ds(128, 128)]),
        (xtab_hbm.at[nwf.at[pl.ds(256, 64)]], neg.at[pl.ds(256, 64)]),
    ]

  def out_pair(slot, c):
    gbase = base + c * CB
    off = pl.multiple_of(gbase * TPW // 128, 8)
    return (slots[slot][5], out_hbm.at[pl.ds(off, orows)])

  def issue(pairs, sem):
    for s, d in pairs:
      pltpu.async_copy(s, d, sem)

  def drain(pairs, sem):
    for s, d in pairs:
      pltpu.make_async_copy(s, d, sem).wait()

  def compute(slot):
    cen, ctx, neg, out = slots[slot][2:6]

    lane15 = lax.iota(jnp.int32, LANES) == (LANES - 1)

    @plsc.parallel_loop(0, CB, unroll=2)
    def _b(b):
      c_regs = [cen[b, pl.ds(k * LANES, LANES)] for k in range(nreg)]
      rowv = jnp.full((LANES,), b // 2, jnp.int32)
      colbase = jnp.full((LANES,), (b % 2) * TPW, jnp.int32)

      def emit(src_ref, r, t):
        terms = [src_ref[r, pl.ds(k * LANES, LANES)] * c_regs[k]
                 for k in range(nreg)]
        while len(terms) > 1:
          terms = [terms[i] + terms[i + 1] for i in range(0, len(terms) - 1, 2)
                   ] + ([terms[-1]] if len(terms) % 2 else [])
        sc = plsc.cumsum(terms[0])
        plsc.store_scatter(out, [rowv, colbase + t], sc, mask=lane15)

      emit(ctx, b, 0)
      for t in range(N):
        emit(neg, b * N + t, 1 + t)

  def step(slot, c):
    other = 1 - slot

    @pl.when(c + 1 < nchunk)
    def _():
      drain([nw_stage_pair(other, c + 1)], sem_i[other])
      repack(other)
      issue(gather_pairs(other, c + 1), sem_g[other])

    @pl.when(c + 2 < nchunk)
    def _():
      issue([nw_stage_pair(slot, c + 2)], sem_i[slot])

    drain(gather_pairs(slot, c), sem_g[slot])

    @pl.when(c >= 2)
    def _():
      drain([out_pair(slot, c - 2)], sem_o[slot])

    compute(slot)
    issue([out_pair(slot, c)], sem_o[slot])

  zeros = jnp.zeros((LANES,), jnp.float32)
  for slot in range(2):
    out = slots[slot][5]
    for r in range(orows):
      for c8 in range(128 // LANES):
        out[r, pl.ds(c8 * LANES, LANES)] = zeros

  idx_prefetch = [
      (cw_hbm.at[pl.ds(base, bpw)], cwa),
      (xw_hbm.at[pl.ds(base, bpw)], xwa),
      nw_stage_pair(0, 0),
  ]
  issue(idx_prefetch, sem_g[0])
  drain(idx_prefetch, sem_g[0])
  repack(0)
  issue(gather_pairs(0, 0), sem_g[0])
  issue([nw_stage_pair(1, 1)], sem_i[1])

  @pl.loop(0, nchunk, step=2)
  def _pair(c0):
    step(0, c0)
    step(1, c0 + 1)

  drain([out_pair(0, nchunk - 2)], sem_o[0])
  drain([out_pair(1, nchunk - 1)], sem_o[1])


def _sc_scores(cw, xw, nw2d, ctab, xtab):
  B = cw.shape[0]
  N = nw2d.shape[1]
  D = ctab.shape[1]
  T = N + 1
  TPW = 64
  orows = CB * TPW // 128
  mesh = plsc.VectorSubcoreMesh(core_axis_name="c", subcore_axis_name="s",
                                num_cores=NC, num_subcores=NS)
  bpw = B // NW
  slot_scr = [
      pltpu.VMEM((CB, N), jnp.int32),
      pltpu.VMEM((CB * N,), jnp.int32),
      pltpu.VMEM((CB, D), jnp.float32),
      pltpu.VMEM((CB, D), jnp.float32),
      pltpu.VMEM((CB * N, D), jnp.float32),
      pltpu.VMEM((orows, 128), jnp.float32),
  ]
  idx_scr = [
      pltpu.VMEM((bpw,), jnp.int32),
      pltpu.VMEM((bpw,), jnp.int32),
  ]
  body = functools.partial(_sc_body, B, N, D)
  f = pl.kernel(
      body,
      out_type=jax.ShapeDtypeStruct((B * TPW // 128, 128), jnp.float32),
      mesh=mesh,
      compiler_params=pltpu.CompilerParams(needs_layout_passes=False),
      scratch_types=(idx_scr + slot_scr + slot_scr
                     + [pltpu.SemaphoreType.DMA] * 6),
  )
  return f(cw, xw, nw2d, ctab, xtab)


def _tc_loss_body(RB, pad_const, inv_b, x_ref, o_ref):
  i = pl.program_id(0)
  s = x_ref[...]
  sp_sum = jnp.sum(jnp.log1p(jnp.exp(s)))
  lane = lax.broadcasted_iota(jnp.int32, s.shape, 1)
  pos_sum = jnp.sum(jnp.where(lane % 64 == 0, s, 0.0))
  psum = (sp_sum - pos_sum) * inv_b

  @pl.when(i == 0)
  def _():
    o_ref[...] = jnp.full_like(o_ref, -pad_const)

  o_ref[...] += psum


def _tc_loss(part, B, T):
  R8 = part.shape[0]
  RB = 2048
  assert R8 % RB == 0
  grid = R8 // RB
  TPW = 64
  pad_const = (TPW - T) * 0.6931471805599453
  body = functools.partial(_tc_loss_body, RB, pad_const, 1.0 / B)
  out = pl.pallas_call(
      body,
      grid=(grid,),
      in_specs=[pl.BlockSpec((RB, 128), lambda i: (i, 0))],
      out_specs=pl.BlockSpec((1, 1), lambda i: (0, 0)),
      out_shape=jax.ShapeDtypeStruct((1, 1), jnp.float32),
  )(part)
  return out[0, 0]


def kernel(center_words, context_words, negative_words, center_table,
           context_table):
  B = center_words.shape[0]
  N = negative_words.shape[1]
  cw = center_words.astype(jnp.int32)
  xw = context_words.astype(jnp.int32)
  nw = negative_words.astype(jnp.int32)
  part = _sc_scores(cw, xw, nw, center_table, context_table)
  return _tc_loss(part, B, N + 1)


# speedup vs baseline: 1.8996x; 1.8996x over previous
import functools

import jax
import jax.numpy as jnp
from jax import lax
from jax.experimental import pallas as pl
from jax.experimental.pallas import tpu as pltpu
from jax.experimental.pallas import tpu_sc as plsc

NC = 2
NS = 16
NW = NC * NS
LANES = 16
CB = 16


def _sc_body(B, N, D, cw_hbm, xw_hbm, nw_hbm, ctab_hbm, xtab_hbm, out_hbm,
             *scr):
  T = N + 1
  TPW = 64
  orows = CB * TPW // 128
  bpw = B // NW
  nchunk = bpw // CB
  nreg = D // LANES
  cwa, xwa = scr[0:2]
  slots = [scr[2:8], scr[8:14]]
  sem_i = scr[14:16]
  sem_g = scr[16:18]
  sem_o = scr[18:20]
  lanes = lax.iota(jnp.int32, LANES)

  wid = lax.axis_index("s") * NC + lax.axis_index("c")
  base = wid * bpw

  def nw_stage_pair(slot, c):
    gbase = base + c * CB
    return (nw_hbm.at[pl.ds(gbase, CB)], slots[slot][0])

  def repack(slot):
    nw2c, nwf = slots[slot][0:2]
    for g in range(CB * N // LANES):
      flat = g * LANES + lanes
      nwf[pl.ds(g * LANES, LANES)] = plsc.load_gather(
          nw2c, [flat // N, flat % N])

  def gather_pairs(slot, c):
    nwf, cen, ctx, neg = slots[slot][1:5]
    cb = c * CB
    return [
        (ctab_hbm.at[cwa.at[pl.ds(cb, CB)]], cen),
        (xtab_hbm.at[xwa.at[pl.ds(cb, CB)]], ctx),
        (xtab_hbm.at[nwf.at[pl.ds(0, 128)]], neg.at[pl.ds(0, 128)]),
        (xtab_hbm.at[nwf.at[pl.ds(128, 128)]], neg.at[pl.ds(128, 128)]),
        (xtab_hbm.at[nwf.at[pl.ds(256, 64)]], neg.at[pl.ds(256, 64)]),
    ]

  def out_pair(slot, c):
    gbase = base + c * CB
    off = pl.multiple_of(gbase * TPW // 128, 8)
    return (slots[slot][5], out_hbm.at[pl.ds(off, orows)])

  def issue(pairs, sem):
    for s, d in pairs:
      pltpu.async_copy(s, d, sem)

  def drain(pairs, sem):
    for s, d in pairs:
      pltpu.make_async_copy(s, d, sem).wait()

  def compute(slot):
    cen, ctx, neg, out = slots[slot][2:6]

    lane15 = lax.iota(jnp.int32, LANES) == (LANES - 1)

    @plsc.parallel_loop(0, CB)
    def _b(b):
      c_regs = [cen[b, pl.ds(k * LANES, LANES)] for k in range(nreg)]
      rowv = jnp.full((LANES,), b // 2, jnp.int32)
      colbase = jnp.full((LANES,), (b % 2) * TPW, jnp.int32)

      def emit(src_ref, r, t):
        terms = [src_ref[r, pl.ds(k * LANES, LANES)] * c_regs[k]
                 for k in range(nreg)]
        while len(terms) > 1:
          terms = [terms[i] + terms[i + 1] for i in range(0, len(terms) - 1, 2)
                   ] + ([terms[-1]] if len(terms) % 2 else [])
        sc = plsc.cumsum(terms[0])
        plsc.store_scatter(out, [rowv, colbase + t], sc, mask=lane15)

      emit(ctx, b, 0)
      for t in range(N):
        emit(neg, b * N + t, 1 + t)

  def step(slot, c):
    other = 1 - slot

    @pl.when(c + 1 < nchunk)
    def _():
      drain([nw_stage_pair(other, c + 1)], sem_i[other])
      repack(other)
      issue(gather_pairs(other, c + 1), sem_g[other])

    @pl.when(c + 2 < nchunk)
    def _():
      issue([nw_stage_pair(slot, c + 2)], sem_i[slot])

    drain(gather_pairs(slot, c), sem_g[slot])

    @pl.when(c >= 2)
    def _():
      drain([out_pair(slot, c - 2)], sem_o[slot])

    compute(slot)
    issue([out_pair(slot, c)], sem_o[slot])

  zeros = jnp.zeros((LANES,), jnp.float32)
  for slot in range(2):
    out = slots[slot][5]
    for r in range(orows):
      for c8 in range(128 // LANES):
        out[r, pl.ds(c8 * LANES, LANES)] = zeros

  idx_prefetch = [
      (cw_hbm.at[pl.ds(base, bpw)], cwa),
      (xw_hbm.at[pl.ds(base, bpw)], xwa),
      nw_stage_pair(0, 0),
  ]
  issue(idx_prefetch, sem_g[0])
  drain(idx_prefetch, sem_g[0])
  repack(0)
  issue(gather_pairs(0, 0), sem_g[0])
  issue([nw_stage_pair(1, 1)], sem_i[1])

  @pl.loop(0, nchunk, step=2)
  def _pair(c0):
    step(0, c0)
    step(1, c0 + 1)

  drain([out_pair(0, nchunk - 2)], sem_o[0])
  drain([out_pair(1, nchunk - 1)], sem_o[1])


def _sc_scores(cw, xw, nw2d, ctab, xtab):
  B = cw.shape[0]
  N = nw2d.shape[1]
  D = ctab.shape[1]
  T = N + 1
  TPW = 64
  orows = CB * TPW // 128
  mesh = plsc.VectorSubcoreMesh(core_axis_name="c", subcore_axis_name="s",
                                num_cores=NC, num_subcores=NS)
  bpw = B // NW
  slot_scr = [
      pltpu.VMEM((CB, N), jnp.int32),
      pltpu.VMEM((CB * N,), jnp.int32),
      pltpu.VMEM((CB, D), jnp.float32),
      pltpu.VMEM((CB, D), jnp.float32),
      pltpu.VMEM((CB * N, D), jnp.float32),
      pltpu.VMEM((orows, 128), jnp.float32),
  ]
  idx_scr = [
      pltpu.VMEM((bpw,), jnp.int32),
      pltpu.VMEM((bpw,), jnp.int32),
  ]
  body = functools.partial(_sc_body, B, N, D)
  f = pl.kernel(
      body,
      out_type=jax.ShapeDtypeStruct((B * TPW // 128, 128), jnp.float32),
      mesh=mesh,
      compiler_params=pltpu.CompilerParams(needs_layout_passes=False),
      scratch_types=(idx_scr + slot_scr + slot_scr
                     + [pltpu.SemaphoreType.DMA] * 6),
  )
  return f(cw, xw, nw2d, ctab, xtab)


def _tc_loss_body(RB, pad_const, inv_b, x_ref, o_ref):
  i = pl.program_id(0)
  s = x_ref[...]
  sp_sum = jnp.sum(jnp.log1p(jnp.exp(s)))
  lane = lax.broadcasted_iota(jnp.int32, s.shape, 1)
  pos_sum = jnp.sum(jnp.where(lane % 64 == 0, s, 0.0))
  psum = (sp_sum - pos_sum) * inv_b

  @pl.when(i == 0)
  def _():
    o_ref[...] = jnp.full_like(o_ref, -pad_const)

  o_ref[...] += psum


def _tc_loss(part, B, T):
  R8 = part.shape[0]
  RB = 2048
  assert R8 % RB == 0
  grid = R8 // RB
  TPW = 64
  pad_const = (TPW - T) * 0.6931471805599453
  body = functools.partial(_tc_loss_body, RB, pad_const, 1.0 / B)
  out = pl.pallas_call(
      body,
      grid=(grid,),
      in_specs=[pl.BlockSpec((RB, 128), lambda i: (i, 0))],
      out_specs=pl.BlockSpec((1, 1), lambda i: (0, 0)),
      out_shape=jax.ShapeDtypeStruct((1, 1), jnp.float32),
  )(part)
  return out[0, 0]


def kernel(center_words, context_words, negative_words, center_table,
           context_table):
  B = center_words.shape[0]
  N = negative_words.shape[1]
  part = _sc_scores(center_words, context_words, negative_words,
                    center_table, context_table)
  return _tc_loss(part, B, N + 1)
